# baseline (speedup 1.0000x reference)
import jax
import jax.numpy as jnp
from jax import lax
from jax.experimental import pallas as pl
from jax.experimental.pallas import tpu as pltpu
from jax.experimental.pallas import tpu_sc as plsc

N_NODES = 10000
N_EDGES = 320000
D = 128
NC = 2
NS = 16
NP = 10240
RPT = NP // NS
NB_A = 157
EA = NS * NB_A * 128
BB = 88
NBUF = 3
CHI = 39
NWIN = 3
NB_B = NWIN * CHI
EB = NC * NS * NB_B * BB
XCH = 64


def _deg_norm_h_body(idx_hbm, x_hbm, h_hbm, norms_hbm,
                     idx_v, ones_v, norm_v, xrow_v, hist_sp, semh):
    c = lax.axis_index("c")
    s = lax.axis_index("s")
    base = s * RPT

    for i in range(RPT // 16):
        norm_v[pl.ds(i * 16, 16)] = jnp.zeros((16,), jnp.float32)
    pltpu.sync_copy(norm_v, hist_sp.at[pl.ds(base, RPT)])
    for i in range(128 // 16):
        ones_v[pl.ds(i * 16, 16)] = jnp.ones((16,), jnp.float32)
    plsc.subcore_barrier()

    pltpu.sync_copy(idx_hbm.at[c, s], idx_v)

    @pl.loop(0, NB_A)
    def _acc(b):
        pltpu.async_copy(ones_v, hist_sp.at[idx_v.at[b]], semh, add=True)

        @pl.when(b >= 16)
        def _w():
            pltpu.make_async_copy(ones_v, hist_sp.at[idx_v.at[0]],
                                  semh).wait()

    @pl.loop(0, 16)
    def _accdrain(b):
        pltpu.make_async_copy(ones_v, hist_sp.at[idx_v.at[0]],
                              semh).wait()

    plsc.subcore_barrier()

    pltpu.sync_copy(hist_sp.at[pl.ds(base, RPT)], norm_v)

    @pl.loop(0, RPT // 16)
    def _norm(i):
        d = jnp.maximum(norm_v[pl.ds(i * 16, 16)], 1.0)
        bits = lax.bitcast_convert_type(d, jnp.int32)
        y = lax.bitcast_convert_type(
            jnp.int32(0x5F3759DF) - lax.shift_right_logical(bits, 1),
            jnp.float32)
        for _ in range(3):
            y = y * (1.5 - 0.5 * d * y * y)
        norm_v[pl.ds(i * 16, 16)] = y

    pltpu.sync_copy(norm_v, norms_hbm.at[c, pl.ds(base, RPT)])

    @pl.when(c == 0)
    def _scale():
        @pl.loop(0, RPT // XCH)
        def _chunk(k):
            rb = base + k * XCH
            pltpu.sync_copy(x_hbm.at[pl.ds(rb, XCH)], xrow_v)

            @pl.loop(0, XCH // 16)
            def _grp(g):
                nv = norm_v[pl.ds(k * XCH + g * 16, 16)]
                for j in range(16):
                    nb = jnp.broadcast_to(nv[j], (16,))
                    r = g * 16 + j
                    for kk in range(D // 16):
                        v = xrow_v[r, pl.ds(kk * 16, 16)]
                        xrow_v[r, pl.ds(kk * 16, 16)] = v * nb

            pltpu.sync_copy(xrow_v, h_hbm.at[pl.ds(rb, XCH)])


def _edge_pass_body(h_hbm, src_hbm, dst_hbm, z_hbm, agg_out,
                    src_v, dst_v, r0, r1, r2, agg_sp, g0, g1, g2, s0, s1, s2):
    c = lax.axis_index("c")
    s = lax.axis_index("s")
    base = s * RPT
    NITER = CHI // NBUF

    def gath(b, r, sem):
        pltpu.async_copy(h_hbm.at[src_v.at[b]], r, sem)

    def wait_gath(r, sem):
        pltpu.make_async_copy(h_hbm.at[src_v.at[0]], r, sem).wait()

    def scat(b, r, sem):
        pltpu.async_copy(r, agg_sp.at[dst_v.at[b]], sem, add=True)

    def wait_scat(r, sem):
        pltpu.make_async_copy(r, agg_sp.at[dst_v.at[0]], sem).wait()

    pltpu.sync_copy(z_hbm, agg_sp.at[pl.ds(base, RPT)])
    plsc.subcore_barrier()

    @pl.loop(0, NWIN)
    def _window(t):
        pltpu.sync_copy(src_hbm.at[c, s, t], src_v)
        pltpu.sync_copy(dst_hbm.at[c, s, t], dst_v)

        @pl.when(t > 0)
        def _drain_prev():
            wait_scat(r2, s2)

        gath(0, r0, g0)
        gath(1, r1, g1)

        @pl.loop(0, NITER)
        def _pipe(i):
            b = i * NBUF
            wait_gath(r0, g0)
            scat(b, r0, s0)

            @pl.when(i > 0)
            def _ws2():
                wait_scat(r2, s2)

            gath(b + 2, r2, g2)

            wait_gath(r1, g1)
            scat(b + 1, r1, s1)
            wait_scat(r0, s0)

            @pl.when(i < NITER - 1)
            def _g0():
                gath(b + 3, r0, g0)

            wait_gath(r2, g2)
            scat(b + 2, r2, s2)
            wait_scat(r1, s1)

            @pl.when(i < NITER - 1)
            def _g1():
                gath(b + 4, r1, g1)

    wait_scat(r2, s2)
    plsc.subcore_barrier()
    pltpu.sync_copy(agg_sp.at[pl.ds(base, RPT)],
                    agg_out.at[c, pl.ds(base, RPT)])


def _final_body(a_ref, n_ref, w_ref, b_ref, o_ref):
    acc = (a_ref[0] + a_ref[1]) * n_ref[...]
    o_ref[...] = (jnp.dot(acc, w_ref[...], preferred_element_type=jnp.float32)
                  + b_ref[...])


def kernel(edge_index, x, W, b):
    src = edge_index[0].astype(jnp.int32)
    dst = edge_index[1].astype(jnp.int32)

    pad_a = EA - N_EDGES
    fill_a = 10016 + jnp.arange(pad_a, dtype=jnp.int32) % (NP - 10016)
    idx_a = jnp.stack([
        jnp.concatenate([src, fill_a]).reshape(NS, NB_A, 128),
        jnp.concatenate([dst, fill_a]).reshape(NS, NB_A, 128),
    ])
    x_pad = jnp.pad(x, ((0, NP - N_NODES), (0, 0)))

    mesh = plsc.VectorSubcoreMesh(core_axis_name="c", subcore_axis_name="s",
                                  num_cores=NC, num_subcores=NS)
    h, norms = pl.kernel(
        _deg_norm_h_body,
        out_type=[jax.ShapeDtypeStruct((NP, D), jnp.float32),
                  jax.ShapeDtypeStruct((NC, NP), jnp.float32)],
        mesh=mesh,
        scratch_types=[
            pltpu.VMEM((NB_A, 128), jnp.int32),
            pltpu.VMEM((128,), jnp.float32),
            pltpu.VMEM((RPT,), jnp.float32),
            pltpu.VMEM((XCH, D), jnp.float32),
            pltpu.VMEM_SHARED((NP,), jnp.float32),
            pltpu.SemaphoreType.DMA,
        ],
    )(idx_a, x_pad)

    pad_b = EB - N_EDGES
    src_fill = N_NODES + jnp.arange(pad_b, dtype=jnp.int32) % (NP - N_NODES)
    dst_fill = jnp.arange(pad_b, dtype=jnp.int32) % NP
    src_b = jnp.concatenate([src, src_fill]).reshape(NC, NS, NWIN, CHI, BB)
    dst_b = jnp.concatenate([dst, dst_fill]).reshape(NC, NS, NWIN, CHI, BB)
    zeros = jnp.zeros((RPT, D), jnp.float32)

    mesh_b = plsc.VectorSubcoreMesh(core_axis_name="c", subcore_axis_name="s",
                                    num_cores=NC, num_subcores=NS)
    aggs = pl.kernel(
        _edge_pass_body,
        out_type=jax.ShapeDtypeStruct((NC, NP, D), jnp.float32),
        mesh=mesh_b,
        scratch_types=[
            pltpu.VMEM((CHI, BB), jnp.int32),
            pltpu.VMEM((CHI, BB), jnp.int32),
            pltpu.VMEM((BB, D), jnp.float32),
            pltpu.VMEM((BB, D), jnp.float32),
            pltpu.VMEM((BB, D), jnp.float32),
            pltpu.VMEM_SHARED((NP, D), jnp.float32),
            pltpu.SemaphoreType.DMA,
            pltpu.SemaphoreType.DMA,
            pltpu.SemaphoreType.DMA,
            pltpu.SemaphoreType.DMA,
            pltpu.SemaphoreType.DMA,
            pltpu.SemaphoreType.DMA,
        ],
    )(h, src_b, dst_b, zeros)

    norm_dst = norms[1].reshape(NP, 1)
    return pl.pallas_call(
        _final_body,
        grid=(N_NODES // 400,),
        in_specs=[pl.BlockSpec((NC, 400, D), lambda i: (0, i, 0)),
                  pl.BlockSpec((400, 1), lambda i: (i, 0)),
                  pl.BlockSpec((D, D), lambda i: (0, 0)),
                  pl.BlockSpec((1, D), lambda i: (0, 0))],
        out_specs=pl.BlockSpec((400, D), lambda i: (i, 0)),
        out_shape=jax.ShapeDtypeStruct((N_NODES, D), jnp.float32),
    )(aggs, norm_dst, W, b.reshape(1, D))


# speedup vs baseline: 1.0056x; 1.0056x over previous
import jax
import jax.numpy as jnp
from jax import lax
from jax.experimental import pallas as pl
from jax.experimental.pallas import tpu as pltpu
from jax.experimental.pallas import tpu_sc as plsc

N_NODES = 10000
N_EDGES = 320000
D = 128
NC = 2
NS = 16
NP = 10240
RPT = NP // NS
NB_A = 157
EA = NS * NB_A * 128
BB = 80
NBUF = 3
CHI = 42
NWIN = 3
NB_B = NWIN * CHI
EB = NC * NS * NB_B * BB
XCH = 64


def _deg_norm_h_body(idx_hbm, x_hbm, h_hbm, norms_hbm,
                     idx_v, ones_v, norm_v, xrow_v, hist_sp, semh):
    c = lax.axis_index("c")
    s = lax.axis_index("s")
    base = s * RPT

    for i in range(RPT // 16):
        norm_v[pl.ds(i * 16, 16)] = jnp.zeros((16,), jnp.float32)
    pltpu.sync_copy(norm_v, hist_sp.at[pl.ds(base, RPT)])
    for i in range(128 // 16):
        ones_v[pl.ds(i * 16, 16)] = jnp.ones((16,), jnp.float32)
    plsc.subcore_barrier()

    pltpu.sync_copy(idx_hbm.at[c, s], idx_v)

    @pl.loop(0, NB_A)
    def _acc(b):
        pltpu.async_copy(ones_v, hist_sp.at[idx_v.at[b]], semh, add=True)

        @pl.when(b >= 16)
        def _w():
            pltpu.make_async_copy(ones_v, hist_sp.at[idx_v.at[0]],
                                  semh).wait()

    @pl.loop(0, 16)
    def _accdrain(b):
        pltpu.make_async_copy(ones_v, hist_sp.at[idx_v.at[0]],
                              semh).wait()

    plsc.subcore_barrier()

    pltpu.sync_copy(hist_sp.at[pl.ds(base, RPT)], norm_v)

    @pl.loop(0, RPT // 16)
    def _norm(i):
        d = jnp.maximum(norm_v[pl.ds(i * 16, 16)], 1.0)
        bits = lax.bitcast_convert_type(d, jnp.int32)
        y = lax.bitcast_convert_type(
            jnp.int32(0x5F3759DF) - lax.shift_right_logical(bits, 1),
            jnp.float32)
        for _ in range(3):
            y = y * (1.5 - 0.5 * d * y * y)
        norm_v[pl.ds(i * 16, 16)] = y

    pltpu.sync_copy(norm_v, norms_hbm.at[c, pl.ds(base, RPT)])

    @pl.when(c == 0)
    def _scale():
        @pl.loop(0, RPT // XCH)
        def _chunk(k):
            rb = base + k * XCH
            pltpu.sync_copy(x_hbm.at[pl.ds(rb, XCH)], xrow_v)

            @pl.loop(0, XCH // 16)
            def _grp(g):
                nv = norm_v[pl.ds(k * XCH + g * 16, 16)]
                for j in range(16):
                    nb = jnp.broadcast_to(nv[j], (16,))
                    r = g * 16 + j
                    for kk in range(D // 16):
                        v = xrow_v[r, pl.ds(kk * 16, 16)]
                        xrow_v[r, pl.ds(kk * 16, 16)] = v * nb

            pltpu.sync_copy(xrow_v, h_hbm.at[pl.ds(rb, XCH)])


def _edge_pass_body(h_hbm, src_hbm, dst_hbm, z_hbm, agg_out,
                    src_v, dst_v, r0, r1, r2, agg_sp, g0, g1, g2, s0, s1, s2):
    c = lax.axis_index("c")
    s = lax.axis_index("s")
    base = s * RPT
    NITER = CHI // NBUF

    def gath(b, r, sem):
        pltpu.async_copy(h_hbm.at[src_v.at[b]], r, sem)

    def wait_gath(r, sem):
        pltpu.make_async_copy(h_hbm.at[src_v.at[0]], r, sem).wait()

    def scat(b, r, sem):
        pltpu.async_copy(r, agg_sp.at[dst_v.at[b]], sem, add=True)

    def wait_scat(r, sem):
        pltpu.make_async_copy(r, agg_sp.at[dst_v.at[0]], sem).wait()

    pltpu.sync_copy(z_hbm, agg_sp.at[pl.ds(base, RPT)])
    plsc.subcore_barrier()

    @pl.loop(0, NWIN)
    def _window(t):
        pltpu.sync_copy(src_hbm.at[c, s, t], src_v)
        pltpu.sync_copy(dst_hbm.at[c, s, t], dst_v)

        @pl.when(t > 0)
        def _drain_prev():
            wait_scat(r2, s2)

        gath(0, r0, g0)
        gath(1, r1, g1)

        @pl.loop(0, NITER)
        def _pipe(i):
            b = i * NBUF
            wait_gath(r0, g0)
            scat(b, r0, s0)

            @pl.when(i > 0)
            def _ws2():
                wait_scat(r2, s2)

            gath(b + 2, r2, g2)

            wait_gath(r1, g1)
            scat(b + 1, r1, s1)
            wait_scat(r0, s0)

            @pl.when(i < NITER - 1)
            def _g0():
                gath(b + 3, r0, g0)

            wait_gath(r2, g2)
            scat(b + 2, r2, s2)
            wait_scat(r1, s1)

            @pl.when(i < NITER - 1)
            def _g1():
                gath(b + 4, r1, g1)

    wait_scat(r2, s2)
    plsc.subcore_barrier()
    pltpu.sync_copy(agg_sp.at[pl.ds(base, RPT)],
                    agg_out.at[c, pl.ds(base, RPT)])


def _final_body(a_ref, n_ref, w_ref, b_ref, o_ref):
    acc = (a_ref[0] + a_ref[1]) * n_ref[...]
    o_ref[...] = (jnp.dot(acc, w_ref[...], preferred_element_type=jnp.float32)
                  + b_ref[...])


def kernel(edge_index, x, W, b):
    src = edge_index[0].astype(jnp.int32)
    dst = edge_index[1].astype(jnp.int32)

    pad_a = EA - N_EDGES
    fill_a = 10016 + jnp.arange(pad_a, dtype=jnp.int32) % (NP - 10016)
    idx_a = jnp.stack([
        jnp.concatenate([src, fill_a]).reshape(NS, NB_A, 128),
        jnp.concatenate([dst, fill_a]).reshape(NS, NB_A, 128),
    ])
    x_pad = jnp.pad(x, ((0, NP - N_NODES), (0, 0)))

    mesh = plsc.VectorSubcoreMesh(core_axis_name="c", subcore_axis_name="s",
                                  num_cores=NC, num_subcores=NS)
    h, norms = pl.kernel(
        _deg_norm_h_body,
        out_type=[jax.ShapeDtypeStruct((NP, D), jnp.float32),
                  jax.ShapeDtypeStruct((NC, NP), jnp.float32)],
        mesh=mesh,
        scratch_types=[
            pltpu.VMEM((NB_A, 128), jnp.int32),
            pltpu.VMEM((128,), jnp.float32),
            pltpu.VMEM((RPT,), jnp.float32),
            pltpu.VMEM((XCH, D), jnp.float32),
            pltpu.VMEM_SHARED((NP,), jnp.float32),
            pltpu.SemaphoreType.DMA,
        ],
    )(idx_a, x_pad)

    pad_b = EB - N_EDGES
    src_fill = N_NODES + jnp.arange(pad_b, dtype=jnp.int32) % (NP - N_NODES)
    dst_fill = jnp.arange(pad_b, dtype=jnp.int32) % NP
    src_b = jnp.concatenate([src, src_fill]).reshape(NC, NS, NWIN, CHI, BB)
    dst_b = jnp.concatenate([dst, dst_fill]).reshape(NC, NS, NWIN, CHI, BB)
    zeros = jnp.zeros((RPT, D), jnp.float32)

    mesh_b = plsc.VectorSubcoreMesh(core_axis_name="c", subcore_axis_name="s",
                                    num_cores=NC, num_subcores=NS)
    aggs = pl.kernel(
        _edge_pass_body,
        out_type=jax.ShapeDtypeStruct((NC, NP, D), jnp.float32),
        mesh=mesh_b,
        scratch_types=[
            pltpu.VMEM((CHI, BB), jnp.int32),
            pltpu.VMEM((CHI, BB), jnp.int32),
            pltpu.VMEM((BB, D), jnp.float32),
            pltpu.VMEM((BB, D), jnp.float32),
            pltpu.VMEM((BB, D), jnp.float32),
            pltpu.VMEM_SHARED((NP, D), jnp.float32),
            pltpu.SemaphoreType.DMA,
            pltpu.SemaphoreType.DMA,
            pltpu.SemaphoreType.DMA,
            pltpu.SemaphoreType.DMA,
            pltpu.SemaphoreType.DMA,
            pltpu.SemaphoreType.DMA,
        ],
    )(h, src_b, dst_b, zeros)

    norm_dst = norms[1].reshape(NP, 1)
    return pl.pallas_call(
        _final_body,
        grid=(N_NODES // 400,),
        in_specs=[pl.BlockSpec((NC, 400, D), lambda i: (0, i, 0)),
                  pl.BlockSpec((400, 1), lambda i: (i, 0)),
                  pl.BlockSpec((D, D), lambda i: (0, 0)),
                  pl.BlockSpec((1, D), lambda i: (0, 0))],
        out_specs=pl.BlockSpec((400, D), lambda i: (i, 0)),
        out_shape=jax.ShapeDtypeStruct((N_NODES, D), jnp.float32),
    )(aggs, norm_dst, W, b.reshape(1, D))
